# baseline (speedup 1.0000x reference)
import functools

import jax
import jax.numpy as jnp
from jax import lax
from jax.experimental import pallas as pl
from jax.experimental.pallas import tpu as pltpu
from jax.experimental.pallas import tpu_sc as plsc

_NC, _NS, _L = 2, 16, 16


def _make_sc_add(nb, seq, d):
    nw = _NC * _NS
    pos_per_w = seq // nw
    c_rows = 16
    nchunk = pos_per_w // c_rows

    mesh = plsc.VectorSubcoreMesh(core_axis_name="c", subcore_axis_name="s")

    def body(x_hbm, pos_hbm, out_hbm, xb, pb, ob, sx, sp, so):
        wid = lax.axis_index("s") * _NC + lax.axis_index("c")
        wbase = wid * pos_per_w
        pairs = [(k, b) for k in range(nchunk) for b in range(nb)]

        def start_pos(k):
            return pltpu.async_copy(
                pos_hbm.at[pl.ds(wbase + k * c_rows, c_rows), :], pb[k & 1], sp[k & 1]
            )

        def start_x(t, bb):
            k, b = pairs[t]
            row = b * seq + wbase + k * c_rows
            return pltpu.async_copy(x_hbm.at[pl.ds(row, c_rows), :], xb[bb], sx[bb])

        hp = [start_pos(0), None]
        hx = [start_x(0, 0), start_x(1, 1) if len(pairs) > 1 else None]
        hout = [None, None]
        for t, (k, b) in enumerate(pairs):
            bb = t & 1
            kk = k & 1
            if b == 0:
                hp[kk].wait()
            hx[bb].wait()
            if hout[bb] is not None:
                hout[bb].wait()

            @plsc.parallel_loop(0, c_rows)
            def _(r):
                @plsc.parallel_loop(0, d, step=_L, unroll=8)
                def _(i):
                    ob[bb][r, pl.ds(i, _L)] = xb[bb][r, pl.ds(i, _L)] + pb[kk][r, pl.ds(i, _L)]

            row = b * seq + wbase + k * c_rows
            hout[bb] = pltpu.async_copy(ob[bb], out_hbm.at[pl.ds(row, c_rows), :], so[bb])
            if t + 2 < len(pairs):
                hx[bb] = start_x(t + 2, bb)
            if b == 0 and k + 1 < nchunk:
                hp[(k + 1) & 1] = start_pos(k + 1)
        for h in hout:
            if h is not None:
                h.wait()

    buf = lambda: [pltpu.VMEM((c_rows, d), jnp.float32) for _ in range(2)]
    sem = lambda: [pltpu.SemaphoreType.DMA for _ in range(2)]
    return pl.kernel(
        body,
        out_type=jax.ShapeDtypeStruct((nb * seq, d), jnp.float32),
        mesh=mesh,
        scratch_types=[buf(), buf(), buf(), sem(), sem(), sem()],
    )


def _tc_body(x_ref, p_ref, o_ref):
    o_ref[...] = x_ref[...] + p_ref[...]


def _tc_add(x, pos, nb, seq, d, b0):
    bs = 256
    nbt = nb - b0
    return pl.pallas_call(
        _tc_body,
        grid=(seq // bs, nbt),
        in_specs=[
            pl.BlockSpec((1, bs, d), lambda s, bt: (b0 + bt, s, 0)),
            pl.BlockSpec((1, bs, d), lambda s, bt: (0, s, 0)),
        ],
        out_specs=pl.BlockSpec((1, bs, d), lambda s, bt: (bt, s, 0)),
        out_shape=jax.ShapeDtypeStruct((nbt, seq, d), x.dtype),
    )(x, pos[None])


def kernel(x, pos_embedding):
    B, S, D = x.shape
    B_SC = 1
    pos = pos_embedding[:S]
    x2 = x.reshape(B * S, D)
    sc_out = _make_sc_add(B_SC, S, D)(x2, pos).reshape(B_SC, S, D)
    tc_out = _tc_add(x, pos, B, S, D, B_SC)
    return jnp.concatenate([sc_out, tc_out], axis=0)


# speedup vs baseline: 1.3605x; 1.3605x over previous
import functools

import jax
import jax.numpy as jnp
from jax import lax
from jax.experimental import pallas as pl
from jax.experimental.pallas import tpu as pltpu
from jax.experimental.pallas import tpu_sc as plsc

_NC, _NS, _L = 2, 16, 16


def _make_sc_add(nb, seq, d):
    nw = _NC * _NS
    pos_per_w = seq // nw
    c_rows = 16
    nchunk = pos_per_w // c_rows
    nring = 3

    mesh = plsc.VectorSubcoreMesh(core_axis_name="c", subcore_axis_name="s")

    def body(x_hbm, pos_hbm, out_hbm, posb, xb, sx, so):
        wid = lax.axis_index("s") * _NC + lax.axis_index("c")
        wbase = wid * pos_per_w
        pltpu.sync_copy(pos_hbm.at[pl.ds(wbase, pos_per_w), :], posb)
        pairs = [(b, k) for b in range(nb) for k in range(nchunk)]

        def start_x(t):
            b, k = pairs[t]
            row = b * seq + wbase + k * c_rows
            bb = t % nring
            return pltpu.async_copy(x_hbm.at[pl.ds(row, c_rows), :], xb[bb], sx[bb])

        hx = [None] * nring
        hout = [None] * nring
        hx[0] = start_x(0)
        hx[1] = start_x(1)
        for t, (b, k) in enumerate(pairs):
            bb = t % nring
            hx[bb].wait()

            @plsc.parallel_loop(0, c_rows)
            def _(r):
                @plsc.parallel_loop(0, d, step=_L, unroll=8)
                def _(i):
                    plsc.addupdate(
                        xb[bb].at[r, pl.ds(i, _L)],
                        posb[k * c_rows + r, pl.ds(i, _L)],
                    )

            row = b * seq + wbase + k * c_rows
            hout[bb] = pltpu.async_copy(xb[bb], out_hbm.at[pl.ds(row, c_rows), :], so[bb])
            if t + 2 < len(pairs):
                bn = (t + 2) % nring
                if hout[bn] is not None:
                    hout[bn].wait()
                hx[bn] = start_x(t + 2)
        for h in hout:
            if h is not None:
                h.wait()

    return pl.kernel(
        body,
        out_type=jax.ShapeDtypeStruct((nb * seq, d), jnp.float32),
        mesh=mesh,
        scratch_types=[
            pltpu.VMEM((pos_per_w, d), jnp.float32),
            [pltpu.VMEM((c_rows, d), jnp.float32) for _ in range(nring)],
            [pltpu.SemaphoreType.DMA for _ in range(nring)],
            [pltpu.SemaphoreType.DMA for _ in range(nring)],
        ],
    )


def kernel(x, pos_embedding):
    B, S, D = x.shape
    x2 = x.reshape(B * S, D)
    out = _make_sc_add(B, S, D)(x2, pos_embedding[:S])
    return out.reshape(B, S, D)


# speedup vs baseline: 1.3792x; 1.0137x over previous
import functools

import jax
import jax.numpy as jnp
from jax import lax
from jax.experimental import pallas as pl
from jax.experimental.pallas import tpu as pltpu
from jax.experimental.pallas import tpu_sc as plsc

_NC, _NS, _L = 2, 16, 16


def _make_sc_add(nb, seq, d):
    nw = _NC * _NS
    pos_per_w = seq // nw
    c_rows = 16
    nchunk = pos_per_w // c_rows
    nring = 3

    mesh = plsc.VectorSubcoreMesh(core_axis_name="c", subcore_axis_name="s")

    def body(x_hbm, pos_hbm, out_hbm, posb, xb, sx, so, spos):
        wid = lax.axis_index("s") * _NC + lax.axis_index("c")
        wbase = wid * pos_per_w
        hpos = pltpu.async_copy(pos_hbm.at[pl.ds(wbase, pos_per_w), :], posb, spos)
        pairs = [(b, k) for b in range(nb) for k in range(nchunk)]

        def start_x(t):
            b, k = pairs[t]
            row = b * seq + wbase + k * c_rows
            bb = t % nring
            return pltpu.async_copy(x_hbm.at[pl.ds(row, c_rows), :], xb[bb], sx[bb])

        hx = [None] * nring
        hout = [None] * nring
        hx[0] = start_x(0)
        hx[1] = start_x(1)
        for t, (b, k) in enumerate(pairs):
            bb = t % nring
            if t == 0:
                hpos.wait()
            hx[bb].wait()

            @plsc.parallel_loop(0, c_rows)
            def _(r):
                @plsc.parallel_loop(0, d, step=_L, unroll=8)
                def _(i):
                    plsc.addupdate(
                        xb[bb].at[r, pl.ds(i, _L)],
                        posb[k * c_rows + r, pl.ds(i, _L)],
                    )

            row = b * seq + wbase + k * c_rows
            hout[bb] = pltpu.async_copy(xb[bb], out_hbm.at[pl.ds(row, c_rows), :], so[bb])
            if t + 2 < len(pairs):
                bn = (t + 2) % nring
                if hout[bn] is not None:
                    hout[bn].wait()
                hx[bn] = start_x(t + 2)
        for h in hout:
            if h is not None:
                h.wait()

    return pl.kernel(
        body,
        out_type=jax.ShapeDtypeStruct((nb * seq, d), jnp.float32),
        mesh=mesh,
        scratch_types=[
            pltpu.VMEM((pos_per_w, d), jnp.float32),
            [pltpu.VMEM((c_rows, d), jnp.float32) for _ in range(nring)],
            [pltpu.SemaphoreType.DMA for _ in range(nring)],
            [pltpu.SemaphoreType.DMA for _ in range(nring)],
            pltpu.SemaphoreType.DMA,
        ],
    )


def kernel(x, pos_embedding):
    B, S, D = x.shape
    x2 = x.reshape(B * S, D)
    out = _make_sc_add(B, S, D)(x2, pos_embedding[:S])
    return out.reshape(B, S, D)


# speedup vs baseline: 1.3996x; 1.0148x over previous
import functools

import jax
import jax.numpy as jnp
from jax import lax
from jax.experimental import pallas as pl
from jax.experimental.pallas import tpu as pltpu
from jax.experimental.pallas import tpu_sc as plsc

_NC, _NS, _L = 2, 16, 16


def _make_sc_add(nb, seq, d):
    nw = _NC * _NS
    pos_per_w = seq // nw
    c_rows = 16
    nchunk = pos_per_w // c_rows
    nring = 3

    mesh = plsc.VectorSubcoreMesh(core_axis_name="c", subcore_axis_name="s")

    def body(x_hbm, pos_hbm, out_hbm, posb, xb, sx, so, spos):
        wid = lax.axis_index("s") * _NC + lax.axis_index("c")
        wbase = wid * pos_per_w
        pairs = [(b, k) for b in range(nb) for k in range(nchunk)]

        def start_x(t):
            b, k = pairs[t]
            row = b * seq + wbase + k * c_rows
            bb = t % nring
            return pltpu.async_copy(x_hbm.at[pl.ds(row, c_rows), :], xb[bb], sx[bb])

        def start_pos(k):
            return pltpu.async_copy(
                pos_hbm.at[pl.ds(wbase + k * c_rows, c_rows), :], posb[k], spos[k]
            )

        hpos = [None] * nchunk
        hpos[0] = start_pos(0)
        hx = [None] * nring
        hout = [None] * nring
        hx[0] = start_x(0)
        hpos[1] = start_pos(1)
        hx[1] = start_x(1)
        for k in range(2, nchunk):
            hpos[k] = start_pos(k)
        for t, (b, k) in enumerate(pairs):
            bb = t % nring
            if b == 0:
                hpos[k].wait()
            hx[bb].wait()

            @plsc.parallel_loop(0, c_rows)
            def _(r):
                @plsc.parallel_loop(0, d, step=_L, unroll=8)
                def _(i):
                    plsc.addupdate(
                        xb[bb].at[r, pl.ds(i, _L)],
                        posb[k][r, pl.ds(i, _L)],
                    )

            row = b * seq + wbase + k * c_rows
            hout[bb] = pltpu.async_copy(xb[bb], out_hbm.at[pl.ds(row, c_rows), :], so[bb])
            if t + 2 < len(pairs):
                bn = (t + 2) % nring
                if hout[bn] is not None:
                    hout[bn].wait()
                hx[bn] = start_x(t + 2)
        for h in hout:
            if h is not None:
                h.wait()

    return pl.kernel(
        body,
        out_type=jax.ShapeDtypeStruct((nb * seq, d), jnp.float32),
        mesh=mesh,
        scratch_types=[
            [pltpu.VMEM((c_rows, d), jnp.float32) for _ in range(nchunk)],
            [pltpu.VMEM((c_rows, d), jnp.float32) for _ in range(nring)],
            [pltpu.SemaphoreType.DMA for _ in range(nring)],
            [pltpu.SemaphoreType.DMA for _ in range(nring)],
            [pltpu.SemaphoreType.DMA for _ in range(nchunk)],
        ],
    )


def kernel(x, pos_embedding):
    B, S, D = x.shape
    x2 = x.reshape(B * S, D)
    out = _make_sc_add(B, S, D)(x2, pos_embedding[:S])
    return out.reshape(B, S, D)
